# baseline (speedup 1.0000x reference)
import functools

import jax
import jax.numpy as jnp
from jax import lax
from jax.experimental import pallas as pl
from jax.experimental.pallas import tpu as pltpu
from jax.experimental.pallas import tpu_sc as plsc

N = 10000
E = 320000
D = 128

NC = 2
NS = 16
NW = NC * NS
CH = 128
NCHUNK = 80
NHALF = NCHUNK // 2
EPW = CH * NCHUNK
E_PAD = EPW * NW
N_PAD = 10240
RPT = N_PAD // NS
BM = 256

_MESH = plsc.VectorSubcoreMesh(core_axis_name="c", subcore_axis_name="s")


@functools.partial(
    pl.kernel,
    out_type=jax.ShapeDtypeStruct((NC, N_PAD, D), jnp.float32),
    mesh=_MESH,
    scratch_types=[
        pltpu.VMEM((NCHUNK, CH), jnp.int32),
        pltpu.VMEM((CH, D), jnp.float32),
        pltpu.VMEM_SHARED((N_PAD, D), jnp.float32),
        pltpu.SemaphoreType.DMA,
    ],
)
def _sc_degree(dst_hbm, zeros_hbm, ones_hbm, out_hbm, idx_v, ones_v, acc_sh, sem):
    c = lax.axis_index("c")
    s = lax.axis_index("s")
    wid = s * NC + c
    r0 = s * RPT
    pltpu.sync_copy(ones_hbm, ones_v)
    pltpu.sync_copy(dst_hbm.at[wid], idx_v)
    pltpu.sync_copy(zeros_hbm.at[pl.ds(r0, RPT)], acc_sh.at[pl.ds(r0, RPT)])
    plsc.subcore_barrier()

    @pl.loop(0, NCHUNK)
    def _(j):
        pltpu.sync_copy(ones_v, acc_sh.at[idx_v.at[j]], add=True)

    plsc.subcore_barrier()
    pltpu.sync_copy(acc_sh.at[pl.ds(r0, RPT)], out_hbm.at[c, pl.ds(r0, RPT)])


@functools.partial(
    pl.kernel,
    out_type=jax.ShapeDtypeStruct((NC, N_PAD, D), jnp.float32),
    mesh=_MESH,
    scratch_types=[
        pltpu.VMEM((NHALF, CH), jnp.int32),
        pltpu.VMEM((NHALF, CH), jnp.int32),
        pltpu.VMEM((CH, D), jnp.float32),
        pltpu.VMEM((CH, D), jnp.float32),
        pltpu.SemaphoreType.DMA,
        pltpu.SemaphoreType.DMA,
        pltpu.VMEM_SHARED((N_PAD, D), jnp.float32),
    ],
)
def _sc_edges(g_hbm, src_hbm, dst_hbm, zeros_hbm, out_hbm,
              si_v, di_v, rows0, rows1, sem0, sem1, acc_sh):
    c = lax.axis_index("c")
    s = lax.axis_index("s")
    wid = s * NC + c
    r0 = s * RPT

    def gather(j, buf, sem):
        pltpu.async_copy(g_hbm.at[si_v.at[j]], buf, sem)

    def gwait(buf, sem):
        pltpu.make_async_copy(g_hbm.at[si_v.at[0]], buf, sem).wait()

    def scat(buf, j):
        pltpu.sync_copy(buf, acc_sh.at[di_v.at[j]], add=True)

    pltpu.sync_copy(zeros_hbm.at[pl.ds(r0, RPT)], acc_sh.at[pl.ds(r0, RPT)])
    plsc.subcore_barrier()

    for h in range(2):
        pltpu.sync_copy(src_hbm.at[wid, h], si_v)
        pltpu.sync_copy(dst_hbm.at[wid, h], di_v)
        gather(0, rows0, sem0)

        @pl.loop(0, NHALF // 2 - 1)
        def _(t):
            j = 2 * t
            gwait(rows0, sem0)
            gather(j + 1, rows1, sem1)
            scat(rows0, j)
            gwait(rows1, sem1)
            gather(j + 2, rows0, sem0)
            scat(rows1, j + 1)

        gwait(rows0, sem0)
        gather(NHALF - 1, rows1, sem1)
        scat(rows0, NHALF - 2)
        gwait(rows1, sem1)
        scat(rows1, NHALF - 1)

    plsc.subcore_barrier()
    pltpu.sync_copy(acc_sh.at[pl.ds(r0, RPT)], out_hbm.at[c, pl.ds(r0, RPT)])


def _dot(a, b):
    return jnp.dot(a, b, preferred_element_type=jnp.float32,
                   precision=lax.Precision.HIGHEST)


def _dinv_of(deg_ref):
    return lax.rsqrt(deg_ref[0, :, 0:1] + deg_ref[1, :, 0:1] + 1.0)


def _mm_body(x_ref, w_ref, o_ref):
    o_ref[...] = _dot(x_ref[...], w_ref[...])


def _prep_body(t_ref, deg_ref, o_ref):
    o_ref[...] = _dinv_of(deg_ref) * t_ref[...]


def _mid_body(acc_ref, g_ref, deg_ref, b_ref, w_ref, o_ref):
    dinv = _dinv_of(deg_ref)
    h = jnp.maximum(dinv * (acc_ref[0] + acc_ref[1] + g_ref[...]) + b_ref[...], 0.0)
    o_ref[...] = dinv * _dot(h, w_ref[...])


def _fin_body(acc_ref, g_ref, deg_ref, b_ref, o_ref):
    dinv = _dinv_of(deg_ref)
    o_ref[...] = dinv * (acc_ref[0] + acc_ref[1] + g_ref[...]) + b_ref[...]


_GRID = (N_PAD // BM,)
_ROWS = pl.BlockSpec((BM, D), lambda i: (i, 0))
_ACC = pl.BlockSpec((NC, BM, D), lambda i: (0, i, 0))
_DEG = pl.BlockSpec((NC, BM, D), lambda i: (0, i, 0))
_WMAT = pl.BlockSpec((D, D), lambda i: (0, 0))
_BVEC = pl.BlockSpec((1, D), lambda i: (0, 0))
_OUTF = jax.ShapeDtypeStruct((N_PAD, D), jnp.float32)

_mm = pl.pallas_call(_mm_body, grid=_GRID, in_specs=[_ROWS, _WMAT],
                     out_specs=_ROWS, out_shape=_OUTF)
_prep = pl.pallas_call(_prep_body, grid=_GRID, in_specs=[_ROWS, _DEG],
                       out_specs=_ROWS, out_shape=_OUTF)
_mid = pl.pallas_call(_mid_body, grid=_GRID,
                      in_specs=[_ACC, _ROWS, _DEG, _BVEC, _WMAT],
                      out_specs=_ROWS, out_shape=_OUTF)
_fin = pl.pallas_call(_fin_body, grid=_GRID,
                      in_specs=[_ACC, _ROWS, _DEG, _BVEC],
                      out_specs=_ROWS, out_shape=_OUTF)


def kernel(x, edge_index, W1, b1, W2, b2):
    src = edge_index[0]
    dst = edge_index[1]
    pad = jnp.full((E_PAD - E,), N, dtype=jnp.int32)
    srcp = jnp.concatenate([src, pad]).reshape(NW, 2, NHALF, CH)
    dstp = jnp.concatenate([dst, pad]).reshape(NW, 2, NHALF, CH)
    dstp_deg = dstp.reshape(NW, NCHUNK, CH)
    x_pad = jnp.pad(x, ((0, N_PAD - N), (0, 0)))
    zeros128 = jnp.zeros((N_PAD, D), jnp.float32)
    ones = jnp.ones((CH, D), jnp.float32)
    b1r = b1.reshape(1, D)
    b2r = b2.reshape(1, D)

    deg = _sc_degree(dstp_deg, zeros128, ones)
    t1 = _mm(x_pad, W1)
    g1 = _prep(t1, deg)
    acc1 = _sc_edges(g1, srcp, dstp, zeros128)
    g2 = _mid(acc1, g1, deg, b1r, W2)
    acc2 = _sc_edges(g2, srcp, dstp, zeros128)
    out = _fin(acc2, g2, deg, b2r)
    return out[:N]


# speedup vs baseline: 1.0789x; 1.0789x over previous
import functools

import jax
import jax.numpy as jnp
from jax import lax
from jax.experimental import pallas as pl
from jax.experimental.pallas import tpu as pltpu
from jax.experimental.pallas import tpu_sc as plsc

N = 10000
E = 320000
D = 128

NC = 2
NS = 16
NW = NC * NS
CH = 128
NCHUNK = 80
NHALF = NCHUNK // 2
EPW = CH * NCHUNK
E_PAD = EPW * NW
GCH = 64
NSEG = 4
GHALF = EPW // GCH // NSEG
N_PAD = 10240
RPT = N_PAD // NS
BM = 256

_MESH = plsc.VectorSubcoreMesh(core_axis_name="c", subcore_axis_name="s")


@functools.partial(
    pl.kernel,
    out_type=jax.ShapeDtypeStruct((NC, N_PAD, D), jnp.float32),
    mesh=_MESH,
    scratch_types=[
        pltpu.VMEM((NCHUNK, CH), jnp.int32),
        pltpu.VMEM((CH, D), jnp.float32),
        pltpu.VMEM_SHARED((N_PAD, D), jnp.float32),
        pltpu.SemaphoreType.DMA,
    ],
)
def _sc_degree(dst_hbm, zeros_hbm, ones_hbm, out_hbm, idx_v, ones_v, acc_sh, sem):
    c = lax.axis_index("c")
    s = lax.axis_index("s")
    wid = s * NC + c
    r0 = s * RPT
    pltpu.sync_copy(ones_hbm, ones_v)
    pltpu.sync_copy(dst_hbm.at[wid], idx_v)
    pltpu.sync_copy(zeros_hbm.at[pl.ds(r0, RPT)], acc_sh.at[pl.ds(r0, RPT)])
    plsc.subcore_barrier()

    @pl.loop(0, NCHUNK)
    def _(j):
        pltpu.sync_copy(ones_v, acc_sh.at[idx_v.at[j]], add=True)

    plsc.subcore_barrier()
    pltpu.sync_copy(acc_sh.at[pl.ds(r0, RPT)], out_hbm.at[c, pl.ds(r0, RPT)])


@functools.partial(
    pl.kernel,
    out_type=jax.ShapeDtypeStruct((NC, N_PAD, D), jnp.float32),
    mesh=_MESH,
    scratch_types=[
        pltpu.VMEM((GHALF, GCH), jnp.int32),
        pltpu.VMEM((GHALF, GCH), jnp.int32),
        pltpu.VMEM((GCH, D), jnp.float32),
        pltpu.VMEM((GCH, D), jnp.float32),
        pltpu.VMEM((GCH, D), jnp.float32),
        pltpu.VMEM((GCH, D), jnp.float32),
        pltpu.SemaphoreType.DMA,
        pltpu.SemaphoreType.DMA,
        pltpu.SemaphoreType.DMA,
        pltpu.SemaphoreType.DMA,
        pltpu.VMEM_SHARED((N_PAD, D), jnp.float32),
    ],
)
def _sc_edges(g_hbm, src_hbm, dst_hbm, zeros_hbm, out_hbm,
              si_v, di_v, b0, b1, b2, b3, s0, s1, s2, s3, acc_sh):
    c = lax.axis_index("c")
    s = lax.axis_index("s")
    wid = s * NC + c
    r0 = s * RPT
    bufs = [b0, b1, b2, b3]
    sems = [s0, s1, s2, s3]

    def gather(j, b):
        pltpu.async_copy(g_hbm.at[si_v.at[j]], bufs[b], sems[b])

    def gwait(b):
        pltpu.make_async_copy(g_hbm.at[si_v.at[0]], bufs[b], sems[b]).wait()

    def scat(j, b):
        pltpu.sync_copy(bufs[b], acc_sh.at[di_v.at[j]], add=True)

    pltpu.sync_copy(zeros_hbm.at[pl.ds(r0, RPT)], acc_sh.at[pl.ds(r0, RPT)])
    plsc.subcore_barrier()

    for h in range(NSEG):
        pltpu.sync_copy(src_hbm.at[wid, h], si_v)
        pltpu.sync_copy(dst_hbm.at[wid, h], di_v)
        for b in range(3):
            gather(b, b)

        @pl.loop(0, GHALF // 4)
        def _(t):
            j = 4 * t
            for b in range(4):
                jj = j + b
                gwait(b)

                @pl.when(jj + 3 < GHALF)
                def _():
                    gather(jj + 3, (b + 3) % 4)

                scat(jj, b)

    plsc.subcore_barrier()
    pltpu.sync_copy(acc_sh.at[pl.ds(r0, RPT)], out_hbm.at[c, pl.ds(r0, RPT)])


def _dot(a, b):
    return jnp.dot(a, b, preferred_element_type=jnp.float32,
                   precision=lax.Precision.HIGHEST)


def _dinv_of(deg_ref):
    return lax.rsqrt(deg_ref[0, :, 0:1] + deg_ref[1, :, 0:1] + 1.0)


def _mm_body(x_ref, w_ref, o_ref):
    o_ref[...] = _dot(x_ref[...], w_ref[...])


def _prep_body(t_ref, deg_ref, o_ref):
    o_ref[...] = _dinv_of(deg_ref) * t_ref[...]


def _mid_body(acc_ref, g_ref, deg_ref, b_ref, w_ref, o_ref):
    dinv = _dinv_of(deg_ref)
    h = jnp.maximum(dinv * (acc_ref[0] + acc_ref[1] + g_ref[...]) + b_ref[...], 0.0)
    o_ref[...] = dinv * _dot(h, w_ref[...])


def _fin_body(acc_ref, g_ref, deg_ref, b_ref, o_ref):
    dinv = _dinv_of(deg_ref)
    o_ref[...] = dinv * (acc_ref[0] + acc_ref[1] + g_ref[...]) + b_ref[...]


_GRID = (N_PAD // BM,)
_ROWS = pl.BlockSpec((BM, D), lambda i: (i, 0))
_ACC = pl.BlockSpec((NC, BM, D), lambda i: (0, i, 0))
_DEG = pl.BlockSpec((NC, BM, D), lambda i: (0, i, 0))
_WMAT = pl.BlockSpec((D, D), lambda i: (0, 0))
_BVEC = pl.BlockSpec((1, D), lambda i: (0, 0))
_OUTF = jax.ShapeDtypeStruct((N_PAD, D), jnp.float32)

_mm = pl.pallas_call(_mm_body, grid=_GRID, in_specs=[_ROWS, _WMAT],
                     out_specs=_ROWS, out_shape=_OUTF)
_prep = pl.pallas_call(_prep_body, grid=_GRID, in_specs=[_ROWS, _DEG],
                       out_specs=_ROWS, out_shape=_OUTF)
_mid = pl.pallas_call(_mid_body, grid=_GRID,
                      in_specs=[_ACC, _ROWS, _DEG, _BVEC, _WMAT],
                      out_specs=_ROWS, out_shape=_OUTF)
_fin = pl.pallas_call(_fin_body, grid=_GRID,
                      in_specs=[_ACC, _ROWS, _DEG, _BVEC],
                      out_specs=_ROWS, out_shape=_OUTF)


def kernel(x, edge_index, W1, b1, W2, b2):
    src = edge_index[0]
    dst = edge_index[1]
    pad = jnp.full((E_PAD - E,), N, dtype=jnp.int32)
    srcp = jnp.concatenate([src, pad]).reshape(NW, NSEG, GHALF, GCH)
    dstp = jnp.concatenate([dst, pad]).reshape(NW, NSEG, GHALF, GCH)
    dstp_deg = dstp.reshape(NW, NCHUNK, CH)
    x_pad = jnp.pad(x, ((0, N_PAD - N), (0, 0)))
    zeros128 = jnp.zeros((N_PAD, D), jnp.float32)
    ones = jnp.ones((CH, D), jnp.float32)
    b1r = b1.reshape(1, D)
    b2r = b2.reshape(1, D)

    deg = _sc_degree(dstp_deg, zeros128, ones)
    t1 = _mm(x_pad, W1)
    g1 = _prep(t1, deg)
    acc1 = _sc_edges(g1, srcp, dstp, zeros128)
    g2 = _mid(acc1, g1, deg, b1r, W2)
    acc2 = _sc_edges(g2, srcp, dstp, zeros128)
    out = _fin(acc2, g2, deg, b2r)
    return out[:N]


# speedup vs baseline: 1.2163x; 1.1274x over previous
import functools

import jax
import jax.numpy as jnp
from jax import lax
from jax.experimental import pallas as pl
from jax.experimental.pallas import tpu as pltpu
from jax.experimental.pallas import tpu_sc as plsc

N = 10000
E = 320000
D = 128

NC = 2
NS = 16
NW = NC * NS
CH = 128
NCHUNK = 80
NHALF = NCHUNK // 2
EPW = CH * NCHUNK
E_PAD = EPW * NW
GCH = 64
GSEG = 32
NSEGT = 10
SEG_FAST = 8
FAST_CORE = 0
N_PAD = 10240
RPT = N_PAD // NS
BM = 256

_MESH = plsc.VectorSubcoreMesh(core_axis_name="c", subcore_axis_name="s")


@functools.partial(
    pl.kernel,
    out_type=jax.ShapeDtypeStruct((NC, N_PAD, D), jnp.float32),
    mesh=_MESH,
    scratch_types=[
        pltpu.VMEM((NCHUNK, CH), jnp.int32),
        pltpu.VMEM((CH, D), jnp.float32),
        pltpu.VMEM_SHARED((N_PAD, D), jnp.float32),
        pltpu.SemaphoreType.DMA,
    ],
)
def _sc_degree(dst_hbm, zeros_hbm, ones_hbm, out_hbm, idx_v, ones_v, acc_sh, sem):
    c = lax.axis_index("c")
    s = lax.axis_index("s")
    wid = s * NC + c
    r0 = s * RPT
    pltpu.sync_copy(ones_hbm, ones_v)
    pltpu.sync_copy(dst_hbm.at[wid], idx_v)
    pltpu.sync_copy(zeros_hbm.at[pl.ds(r0, RPT)], acc_sh.at[pl.ds(r0, RPT)])
    plsc.subcore_barrier()

    @pl.loop(0, NCHUNK)
    def _(j):
        pltpu.sync_copy(ones_v, acc_sh.at[idx_v.at[j]], add=True)

    plsc.subcore_barrier()
    pltpu.sync_copy(acc_sh.at[pl.ds(r0, RPT)], out_hbm.at[c, pl.ds(r0, RPT)])


@functools.partial(
    pl.kernel,
    out_type=jax.ShapeDtypeStruct((NC, N_PAD, D), jnp.float32),
    mesh=_MESH,
    scratch_types=[
        pltpu.VMEM((GSEG, GCH), jnp.int32),
        pltpu.VMEM((GSEG, GCH), jnp.int32),
        pltpu.VMEM((GCH, D), jnp.float32),
        pltpu.VMEM((GCH, D), jnp.float32),
        pltpu.VMEM((GCH, D), jnp.float32),
        pltpu.VMEM((GCH, D), jnp.float32),
        pltpu.SemaphoreType.DMA,
        pltpu.SemaphoreType.DMA,
        pltpu.SemaphoreType.DMA,
        pltpu.SemaphoreType.DMA,
        pltpu.VMEM_SHARED((N_PAD, D), jnp.float32),
    ],
)
def _sc_edges(g_hbm, src_hbm, dst_hbm, zeros_hbm, out_hbm,
              si_v, di_v, b0, b1, b2, b3, s0, s1, s2, s3, acc_sh):
    c = lax.axis_index("c")
    s = lax.axis_index("s")
    r0 = s * RPT
    bufs = [b0, b1, b2, b3]
    sems = [s0, s1, s2, s3]

    def gather(j, b):
        pltpu.async_copy(g_hbm.at[si_v.at[j]], bufs[b], sems[b])

    def gwait(b):
        pltpu.make_async_copy(g_hbm.at[si_v.at[0]], bufs[b], sems[b]).wait()

    def scat(j, b):
        pltpu.sync_copy(bufs[b], acc_sh.at[di_v.at[j]], add=True)

    def seg(h):
        pltpu.sync_copy(src_hbm.at[s, h], si_v)
        pltpu.sync_copy(dst_hbm.at[s, h], di_v)
        for b in range(3):
            gather(b, b)

        @pl.loop(0, GSEG // 4)
        def _(t):
            j = 4 * t
            for b in range(4):
                jj = j + b
                gwait(b)

                @pl.when(jj + 3 < GSEG)
                def _():
                    gather(jj + 3, (b + 3) % 4)

                scat(jj, b)

    pltpu.sync_copy(zeros_hbm.at[pl.ds(r0, RPT)], acc_sh.at[pl.ds(r0, RPT)])
    plsc.subcore_barrier()

    for h in range(SEG_FAST):
        @pl.when(c == FAST_CORE)
        def _():
            seg(h)

    for h in range(SEG_FAST, NSEGT):
        @pl.when(c != FAST_CORE)
        def _():
            seg(h)

    plsc.subcore_barrier()
    pltpu.sync_copy(acc_sh.at[pl.ds(r0, RPT)], out_hbm.at[c, pl.ds(r0, RPT)])


def _dot(a, b):
    return jnp.dot(a, b, preferred_element_type=jnp.float32,
                   precision=lax.Precision.HIGHEST)


def _dinv_of(deg_ref):
    return lax.rsqrt(deg_ref[0, :, 0:1] + deg_ref[1, :, 0:1] + 1.0)


def _mm_body(x_ref, w_ref, o_ref):
    o_ref[...] = _dot(x_ref[...], w_ref[...])


def _prep_body(t_ref, deg_ref, o_ref):
    o_ref[...] = _dinv_of(deg_ref) * t_ref[...]


def _mid_body(acc_ref, g_ref, deg_ref, b_ref, w_ref, o_ref):
    dinv = _dinv_of(deg_ref)
    h = jnp.maximum(dinv * (acc_ref[0] + acc_ref[1] + g_ref[...]) + b_ref[...], 0.0)
    o_ref[...] = dinv * _dot(h, w_ref[...])


def _fin_body(acc_ref, g_ref, deg_ref, b_ref, o_ref):
    dinv = _dinv_of(deg_ref)
    o_ref[...] = dinv * (acc_ref[0] + acc_ref[1] + g_ref[...]) + b_ref[...]


_GRID = (N_PAD // BM,)
_ROWS = pl.BlockSpec((BM, D), lambda i: (i, 0))
_ACC = pl.BlockSpec((NC, BM, D), lambda i: (0, i, 0))
_DEG = pl.BlockSpec((NC, BM, D), lambda i: (0, i, 0))
_WMAT = pl.BlockSpec((D, D), lambda i: (0, 0))
_BVEC = pl.BlockSpec((1, D), lambda i: (0, 0))
_OUTF = jax.ShapeDtypeStruct((N_PAD, D), jnp.float32)

_mm = pl.pallas_call(_mm_body, grid=_GRID, in_specs=[_ROWS, _WMAT],
                     out_specs=_ROWS, out_shape=_OUTF)
_prep = pl.pallas_call(_prep_body, grid=_GRID, in_specs=[_ROWS, _DEG],
                       out_specs=_ROWS, out_shape=_OUTF)
_mid = pl.pallas_call(_mid_body, grid=_GRID,
                      in_specs=[_ACC, _ROWS, _DEG, _BVEC, _WMAT],
                      out_specs=_ROWS, out_shape=_OUTF)
_fin = pl.pallas_call(_fin_body, grid=_GRID,
                      in_specs=[_ACC, _ROWS, _DEG, _BVEC],
                      out_specs=_ROWS, out_shape=_OUTF)


def kernel(x, edge_index, W1, b1, W2, b2):
    src = edge_index[0]
    dst = edge_index[1]
    pad = jnp.full((E_PAD - E,), N, dtype=jnp.int32)
    srcp = jnp.concatenate([src, pad]).reshape(NS, NSEGT, GSEG, GCH)
    dstp = jnp.concatenate([dst, pad]).reshape(NS, NSEGT, GSEG, GCH)
    dstp_deg = dstp.reshape(NW, NCHUNK, CH)
    x_pad = jnp.pad(x, ((0, N_PAD - N), (0, 0)))
    zeros128 = jnp.zeros((N_PAD, D), jnp.float32)
    ones = jnp.ones((CH, D), jnp.float32)
    b1r = b1.reshape(1, D)
    b2r = b2.reshape(1, D)

    deg = _sc_degree(dstp_deg, zeros128, ones)
    t1 = _mm(x_pad, W1)
    g1 = _prep(t1, deg)
    acc1 = _sc_edges(g1, srcp, dstp, zeros128)
    g2 = _mid(acc1, g1, deg, b1r, W2)
    acc2 = _sc_edges(g2, srcp, dstp, zeros128)
    out = _fin(acc2, g2, deg, b2r)
    return out[:N]


# speedup vs baseline: 1.2164x; 1.0001x over previous
import functools

import jax
import jax.numpy as jnp
from jax import lax
from jax.experimental import pallas as pl
from jax.experimental.pallas import tpu as pltpu
from jax.experimental.pallas import tpu_sc as plsc

N = 10000
E = 320000
D = 128

NC = 2
NS = 16
NW = NC * NS
CH = 128
NCHUNK = 80
NHALF = NCHUNK // 2
EPW = CH * NCHUNK
E_PAD = EPW * NW
GCH = 64
GSEG = 32
NSEGT = 10
SEG_FAST = 8
FAST_CORE = 1
N_PAD = 10240
RPT = N_PAD // NS
BM = 256

_MESH = plsc.VectorSubcoreMesh(core_axis_name="c", subcore_axis_name="s")


@functools.partial(
    pl.kernel,
    out_type=jax.ShapeDtypeStruct((NC, N_PAD, D), jnp.float32),
    mesh=_MESH,
    scratch_types=[
        pltpu.VMEM((NCHUNK, CH), jnp.int32),
        pltpu.VMEM((CH, D), jnp.float32),
        pltpu.VMEM_SHARED((N_PAD, D), jnp.float32),
        pltpu.SemaphoreType.DMA,
    ],
)
def _sc_degree(dst_hbm, zeros_hbm, ones_hbm, out_hbm, idx_v, ones_v, acc_sh, sem):
    c = lax.axis_index("c")
    s = lax.axis_index("s")
    wid = s * NC + c
    r0 = s * RPT
    pltpu.sync_copy(ones_hbm, ones_v)
    pltpu.sync_copy(dst_hbm.at[wid], idx_v)
    pltpu.sync_copy(zeros_hbm.at[pl.ds(r0, RPT)], acc_sh.at[pl.ds(r0, RPT)])
    plsc.subcore_barrier()

    @pl.loop(0, NCHUNK)
    def _(j):
        pltpu.sync_copy(ones_v, acc_sh.at[idx_v.at[j]], add=True)

    plsc.subcore_barrier()
    pltpu.sync_copy(acc_sh.at[pl.ds(r0, RPT)], out_hbm.at[c, pl.ds(r0, RPT)])


@functools.partial(
    pl.kernel,
    out_type=jax.ShapeDtypeStruct((NC, N_PAD, D), jnp.float32),
    mesh=_MESH,
    scratch_types=[
        pltpu.VMEM((GSEG, GCH), jnp.int32),
        pltpu.VMEM((GSEG, GCH), jnp.int32),
        pltpu.VMEM((GCH, D), jnp.float32),
        pltpu.VMEM((GCH, D), jnp.float32),
        pltpu.VMEM((GCH, D), jnp.float32),
        pltpu.VMEM((GCH, D), jnp.float32),
        pltpu.SemaphoreType.DMA,
        pltpu.SemaphoreType.DMA,
        pltpu.SemaphoreType.DMA,
        pltpu.SemaphoreType.DMA,
        pltpu.VMEM_SHARED((N_PAD, D), jnp.float32),
    ],
)
def _sc_edges(g_hbm, src_hbm, dst_hbm, zeros_hbm, out_hbm,
              si_v, di_v, b0, b1, b2, b3, s0, s1, s2, s3, acc_sh):
    c = lax.axis_index("c")
    s = lax.axis_index("s")
    r0 = s * RPT
    bufs = [b0, b1, b2, b3]
    sems = [s0, s1, s2, s3]

    def gather(j, b):
        pltpu.async_copy(g_hbm.at[si_v.at[j]], bufs[b], sems[b])

    def gwait(b):
        pltpu.make_async_copy(g_hbm.at[si_v.at[0]], bufs[b], sems[b]).wait()

    def scat(j, b):
        pltpu.sync_copy(bufs[b], acc_sh.at[di_v.at[j]], add=True)

    def seg(h):
        pltpu.sync_copy(src_hbm.at[s, h], si_v)
        pltpu.sync_copy(dst_hbm.at[s, h], di_v)
        for b in range(3):
            gather(b, b)

        @pl.loop(0, GSEG // 4)
        def _(t):
            j = 4 * t
            for b in range(4):
                jj = j + b
                gwait(b)

                @pl.when(jj + 3 < GSEG)
                def _():
                    gather(jj + 3, (b + 3) % 4)

                scat(jj, b)

    pltpu.sync_copy(zeros_hbm.at[pl.ds(r0, RPT)], acc_sh.at[pl.ds(r0, RPT)])
    plsc.subcore_barrier()

    for h in range(SEG_FAST):
        @pl.when(c == FAST_CORE)
        def _():
            seg(h)

    for h in range(SEG_FAST, NSEGT):
        @pl.when(c != FAST_CORE)
        def _():
            seg(h)

    plsc.subcore_barrier()
    pltpu.sync_copy(acc_sh.at[pl.ds(r0, RPT)], out_hbm.at[c, pl.ds(r0, RPT)])


def _dot(a, b):
    return jnp.dot(a, b, preferred_element_type=jnp.float32,
                   precision=lax.Precision.HIGHEST)


def _dinv_of(deg_ref):
    return lax.rsqrt(deg_ref[0, :, 0:1] + deg_ref[1, :, 0:1] + 1.0)


def _mm_body(x_ref, w_ref, o_ref):
    o_ref[...] = _dot(x_ref[...], w_ref[...])


def _prep_body(t_ref, deg_ref, o_ref):
    o_ref[...] = _dinv_of(deg_ref) * t_ref[...]


def _mid_body(acc_ref, g_ref, deg_ref, b_ref, w_ref, o_ref):
    dinv = _dinv_of(deg_ref)
    h = jnp.maximum(dinv * (acc_ref[0] + acc_ref[1] + g_ref[...]) + b_ref[...], 0.0)
    o_ref[...] = dinv * _dot(h, w_ref[...])


def _fin_body(acc_ref, g_ref, deg_ref, b_ref, o_ref):
    dinv = _dinv_of(deg_ref)
    o_ref[...] = dinv * (acc_ref[0] + acc_ref[1] + g_ref[...]) + b_ref[...]


_GRID = (N_PAD // BM,)
_ROWS = pl.BlockSpec((BM, D), lambda i: (i, 0))
_ACC = pl.BlockSpec((NC, BM, D), lambda i: (0, i, 0))
_DEG = pl.BlockSpec((NC, BM, D), lambda i: (0, i, 0))
_WMAT = pl.BlockSpec((D, D), lambda i: (0, 0))
_BVEC = pl.BlockSpec((1, D), lambda i: (0, 0))
_OUTF = jax.ShapeDtypeStruct((N_PAD, D), jnp.float32)

_mm = pl.pallas_call(_mm_body, grid=_GRID, in_specs=[_ROWS, _WMAT],
                     out_specs=_ROWS, out_shape=_OUTF)
_prep = pl.pallas_call(_prep_body, grid=_GRID, in_specs=[_ROWS, _DEG],
                       out_specs=_ROWS, out_shape=_OUTF)
_mid = pl.pallas_call(_mid_body, grid=_GRID,
                      in_specs=[_ACC, _ROWS, _DEG, _BVEC, _WMAT],
                      out_specs=_ROWS, out_shape=_OUTF)
_fin = pl.pallas_call(_fin_body, grid=_GRID,
                      in_specs=[_ACC, _ROWS, _DEG, _BVEC],
                      out_specs=_ROWS, out_shape=_OUTF)


def kernel(x, edge_index, W1, b1, W2, b2):
    src = edge_index[0]
    dst = edge_index[1]
    pad = jnp.full((E_PAD - E,), N, dtype=jnp.int32)
    srcp = jnp.concatenate([src, pad]).reshape(NS, NSEGT, GSEG, GCH)
    dstp = jnp.concatenate([dst, pad]).reshape(NS, NSEGT, GSEG, GCH)
    dstp_deg = dstp.reshape(NW, NCHUNK, CH)
    x_pad = jnp.pad(x, ((0, N_PAD - N), (0, 0)))
    zeros128 = jnp.zeros((N_PAD, D), jnp.float32)
    ones = jnp.ones((CH, D), jnp.float32)
    b1r = b1.reshape(1, D)
    b2r = b2.reshape(1, D)

    deg = _sc_degree(dstp_deg, zeros128, ones)
    t1 = _mm(x_pad, W1)
    g1 = _prep(t1, deg)
    acc1 = _sc_edges(g1, srcp, dstp, zeros128)
    g2 = _mid(acc1, g1, deg, b1r, W2)
    acc2 = _sc_edges(g2, srcp, dstp, zeros128)
    out = _fin(acc2, g2, deg, b2r)
    return out[:N]


# speedup vs baseline: 1.3673x; 1.1241x over previous
import functools

import jax
import jax.numpy as jnp
from jax import lax
from jax.experimental import pallas as pl
from jax.experimental.pallas import tpu as pltpu
from jax.experimental.pallas import tpu_sc as plsc

N = 10000
E = 320000
D = 128

NC = 2
NS = 16
NW = NC * NS
CH = 128
NCHUNK = 80
NHALF = NCHUNK // 2
EPW = CH * NCHUNK
E_PAD = EPW * NW
GCH = 64
GSEG = 32
NSEGT = 10
SEG_FAST = 8
FAST_CORE = 1
N_PAD = 10240
RPT = N_PAD // NS
BM = 256

_MESH = plsc.VectorSubcoreMesh(core_axis_name="c", subcore_axis_name="s")


@functools.partial(
    pl.kernel,
    out_type=jax.ShapeDtypeStruct((NC, N_PAD, D), jnp.float32),
    mesh=_MESH,
    scratch_types=[
        pltpu.VMEM((NCHUNK, CH), jnp.int32),
        pltpu.VMEM((CH, D), jnp.float32),
        pltpu.VMEM_SHARED((N_PAD, D), jnp.float32),
        pltpu.SemaphoreType.DMA,
    ],
)
def _sc_degree(dst_hbm, zeros_hbm, ones_hbm, out_hbm, idx_v, ones_v, acc_sh, sem):
    c = lax.axis_index("c")
    s = lax.axis_index("s")
    wid = s * NC + c
    r0 = s * RPT
    pltpu.sync_copy(ones_hbm, ones_v)
    pltpu.sync_copy(dst_hbm.at[wid], idx_v)
    pltpu.sync_copy(zeros_hbm.at[pl.ds(r0, RPT)], acc_sh.at[pl.ds(r0, RPT)])
    plsc.subcore_barrier()

    @pl.loop(0, NCHUNK)
    def _(j):
        pltpu.sync_copy(ones_v, acc_sh.at[idx_v.at[j]], add=True)

    plsc.subcore_barrier()
    pltpu.sync_copy(acc_sh.at[pl.ds(r0, RPT)], out_hbm.at[c, pl.ds(r0, RPT)])


@functools.partial(
    pl.kernel,
    out_type=jax.ShapeDtypeStruct((NC, N_PAD, D), jnp.float32),
    mesh=_MESH,
    scratch_types=[
        pltpu.VMEM((GSEG, GCH), jnp.int32),
        pltpu.VMEM((GSEG, GCH), jnp.int32),
        pltpu.VMEM((GCH, D), jnp.float32),
        pltpu.VMEM((GCH, D), jnp.float32),
        pltpu.VMEM((GCH, D), jnp.float32),
        pltpu.VMEM((GCH, D), jnp.float32),
        pltpu.SemaphoreType.DMA,
        pltpu.SemaphoreType.DMA,
        pltpu.SemaphoreType.DMA,
        pltpu.SemaphoreType.DMA,
        pltpu.VMEM_SHARED((N_PAD, D), jnp.float32),
    ],
)
def _sc_edges(ga_hbm, gb_hbm, src_hbm, dst_hbm, zeros_hbm, out_hbm,
              si_v, di_v, b0, b1, b2, b3, s0, s1, s2, s3, acc_sh):
    c = lax.axis_index("c")
    s = lax.axis_index("s")
    r0 = s * RPT
    bufs = [b0, b1, b2, b3]
    sems = [s0, s1, s2, s3]

    def gather(g_hbm, j, b):
        pltpu.async_copy(g_hbm.at[si_v.at[j]], bufs[b], sems[b])

    def gwait(g_hbm, b):
        pltpu.make_async_copy(g_hbm.at[si_v.at[0]], bufs[b], sems[b]).wait()

    def scat(j, b):
        pltpu.sync_copy(bufs[b], acc_sh.at[di_v.at[j]], add=True)

    def seg(g_hbm, h):
        pltpu.sync_copy(src_hbm.at[s, h], si_v)
        pltpu.sync_copy(dst_hbm.at[s, h], di_v)
        for b in range(3):
            gather(g_hbm, b, b)

        @pl.loop(0, GSEG // 4)
        def _(t):
            j = 4 * t
            for b in range(4):
                jj = j + b
                gwait(g_hbm, b)

                @pl.when(jj + 3 < GSEG)
                def _():
                    gather(g_hbm, jj + 3, (b + 3) % 4)

                scat(jj, b)

    pltpu.sync_copy(zeros_hbm.at[pl.ds(r0, RPT)], acc_sh.at[pl.ds(r0, RPT)])
    plsc.subcore_barrier()

    for h in range(SEG_FAST):
        @pl.when(c == FAST_CORE)
        def _():
            seg(ga_hbm, h)

    for h in range(SEG_FAST, NSEGT):
        @pl.when(c != FAST_CORE)
        def _():
            seg(gb_hbm, h)

    plsc.subcore_barrier()
    pltpu.sync_copy(acc_sh.at[pl.ds(r0, RPT)], out_hbm.at[c, pl.ds(r0, RPT)])


def _dot(a, b):
    return jnp.dot(a, b, preferred_element_type=jnp.float32,
                   precision=lax.Precision.HIGHEST)


def _dinv_of(deg_ref):
    return lax.rsqrt(deg_ref[0, :, 0:1] + deg_ref[1, :, 0:1] + 1.0)


def _mm_body(x_ref, w_ref, o_ref):
    o_ref[...] = _dot(x_ref[...], w_ref[...])


def _prep_body(t_ref, deg_ref, o_ref, o2_ref):
    g = _dinv_of(deg_ref) * t_ref[...]
    o_ref[...] = g
    o2_ref[...] = g


def _mid_body(acc_ref, g_ref, deg_ref, b_ref, w_ref, o_ref, o2_ref):
    dinv = _dinv_of(deg_ref)
    h = jnp.maximum(dinv * (acc_ref[0] + acc_ref[1] + g_ref[...]) + b_ref[...], 0.0)
    g2 = dinv * _dot(h, w_ref[...])
    o_ref[...] = g2
    o2_ref[...] = g2


def _fin_body(acc_ref, g_ref, deg_ref, b_ref, o_ref):
    dinv = _dinv_of(deg_ref)
    o_ref[...] = dinv * (acc_ref[0] + acc_ref[1] + g_ref[...]) + b_ref[...]


_GRID = (N_PAD // BM,)
_ROWS = pl.BlockSpec((BM, D), lambda i: (i, 0))
_ACC = pl.BlockSpec((NC, BM, D), lambda i: (0, i, 0))
_DEG = pl.BlockSpec((NC, BM, D), lambda i: (0, i, 0))
_WMAT = pl.BlockSpec((D, D), lambda i: (0, 0))
_BVEC = pl.BlockSpec((1, D), lambda i: (0, 0))
_OUTF = jax.ShapeDtypeStruct((N_PAD, D), jnp.float32)

_mm = pl.pallas_call(_mm_body, grid=_GRID, in_specs=[_ROWS, _WMAT],
                     out_specs=_ROWS, out_shape=_OUTF)
_prep = pl.pallas_call(_prep_body, grid=_GRID, in_specs=[_ROWS, _DEG],
                       out_specs=[_ROWS, _ROWS], out_shape=[_OUTF, _OUTF])
_mid = pl.pallas_call(_mid_body, grid=_GRID,
                      in_specs=[_ACC, _ROWS, _DEG, _BVEC, _WMAT],
                      out_specs=[_ROWS, _ROWS], out_shape=[_OUTF, _OUTF])
_fin = pl.pallas_call(_fin_body, grid=_GRID,
                      in_specs=[_ACC, _ROWS, _DEG, _BVEC],
                      out_specs=_ROWS, out_shape=_OUTF)


def kernel(x, edge_index, W1, b1, W2, b2):
    src = edge_index[0]
    dst = edge_index[1]
    pad = jnp.full((E_PAD - E,), N, dtype=jnp.int32)
    srcp = jnp.concatenate([src, pad]).reshape(NS, NSEGT, GSEG, GCH)
    dstp = jnp.concatenate([dst, pad]).reshape(NS, NSEGT, GSEG, GCH)
    dstp_deg = dstp.reshape(NW, NCHUNK, CH)
    x_pad = jnp.pad(x, ((0, N_PAD - N), (0, 0)))
    zeros128 = jnp.zeros((N_PAD, D), jnp.float32)
    ones = jnp.ones((CH, D), jnp.float32)
    b1r = b1.reshape(1, D)
    b2r = b2.reshape(1, D)

    deg = _sc_degree(dstp_deg, zeros128, ones)
    t1 = _mm(x_pad, W1)
    g1a, g1b = _prep(t1, deg)
    acc1 = _sc_edges(g1a, g1b, srcp, dstp, zeros128)
    g2a, g2b = _mid(acc1, g1a, deg, b1r, W2)
    acc2 = _sc_edges(g2a, g2b, srcp, dstp, zeros128)
    out = _fin(acc2, g2a, deg, b2r)
    return out[:N]


# speedup vs baseline: 1.4140x; 1.0341x over previous
import functools

import jax
import jax.numpy as jnp
from jax import lax
from jax.experimental import pallas as pl
from jax.experimental.pallas import tpu as pltpu
from jax.experimental.pallas import tpu_sc as plsc

N = 10000
E = 320000
D = 128

NC = 2
NS = 16
NW = NC * NS
CH = 128
NCHUNK = 80
NHALF = NCHUNK // 2
EPW = CH * NCHUNK
E_PAD = EPW * NW
GCH = 64
GSEG = 16
NSEGT = 20
SEG_FAST = 15
FAST_CORE = 1
N_PAD = 10240
RPT = N_PAD // NS
BM = 256

_MESH = plsc.VectorSubcoreMesh(core_axis_name="c", subcore_axis_name="s")


@functools.partial(
    pl.kernel,
    out_type=jax.ShapeDtypeStruct((NC, N_PAD, D), jnp.float32),
    mesh=_MESH,
    scratch_types=[
        pltpu.VMEM((NCHUNK, CH), jnp.int32),
        pltpu.VMEM((CH, D), jnp.float32),
        pltpu.VMEM_SHARED((N_PAD, D), jnp.float32),
        pltpu.SemaphoreType.DMA,
    ],
)
def _sc_degree(dst_hbm, zeros_hbm, ones_hbm, out_hbm, idx_v, ones_v, acc_sh, sem):
    c = lax.axis_index("c")
    s = lax.axis_index("s")
    wid = s * NC + c
    r0 = s * RPT
    pltpu.sync_copy(ones_hbm, ones_v)
    pltpu.sync_copy(dst_hbm.at[wid], idx_v)
    pltpu.sync_copy(zeros_hbm.at[pl.ds(r0, RPT)], acc_sh.at[pl.ds(r0, RPT)])
    plsc.subcore_barrier()

    @pl.loop(0, NCHUNK)
    def _(j):
        pltpu.sync_copy(ones_v, acc_sh.at[idx_v.at[j]], add=True)

    plsc.subcore_barrier()
    pltpu.sync_copy(acc_sh.at[pl.ds(r0, RPT)], out_hbm.at[c, pl.ds(r0, RPT)])


@functools.partial(
    pl.kernel,
    out_type=jax.ShapeDtypeStruct((NC, N_PAD, D), jnp.float32),
    mesh=_MESH,
    scratch_types=[
        pltpu.VMEM((GSEG, GCH), jnp.int32),
        pltpu.VMEM((GSEG, GCH), jnp.int32),
        pltpu.VMEM((GCH, D), jnp.float32),
        pltpu.VMEM((GCH, D), jnp.float32),
        pltpu.VMEM((GCH, D), jnp.float32),
        pltpu.VMEM((GCH, D), jnp.float32),
        pltpu.SemaphoreType.DMA,
        pltpu.SemaphoreType.DMA,
        pltpu.SemaphoreType.DMA,
        pltpu.SemaphoreType.DMA,
        pltpu.VMEM_SHARED((N_PAD, D), jnp.float32),
    ],
)
def _sc_edges(ga_hbm, gb_hbm, src_hbm, dst_hbm, zeros_hbm, out_hbm,
              si_v, di_v, b0, b1, b2, b3, s0, s1, s2, s3, acc_sh):
    c = lax.axis_index("c")
    s = lax.axis_index("s")
    r0 = s * RPT
    bufs = [b0, b1, b2, b3]
    sems = [s0, s1, s2, s3]

    def gather(g_hbm, j, b):
        pltpu.async_copy(g_hbm.at[si_v.at[j]], bufs[b], sems[b])

    def gwait(g_hbm, b):
        pltpu.make_async_copy(g_hbm.at[si_v.at[0]], bufs[b], sems[b]).wait()

    def scat(j, b):
        pltpu.sync_copy(bufs[b], acc_sh.at[di_v.at[j]], add=True)

    def seg(g_hbm, h):
        pltpu.sync_copy(src_hbm.at[s, h], si_v)
        pltpu.sync_copy(dst_hbm.at[s, h], di_v)
        for b in range(3):
            gather(g_hbm, b, b)

        @pl.loop(0, GSEG // 4)
        def _(t):
            j = 4 * t
            for b in range(4):
                jj = j + b
                gwait(g_hbm, b)

                @pl.when(jj + 3 < GSEG)
                def _():
                    gather(g_hbm, jj + 3, (b + 3) % 4)

                scat(jj, b)

    pltpu.sync_copy(zeros_hbm.at[pl.ds(r0, RPT)], acc_sh.at[pl.ds(r0, RPT)])
    plsc.subcore_barrier()

    for h in range(SEG_FAST):
        @pl.when(c == FAST_CORE)
        def _():
            seg(ga_hbm, h)

    for h in range(SEG_FAST, NSEGT):
        @pl.when(c != FAST_CORE)
        def _():
            seg(gb_hbm, h)

    plsc.subcore_barrier()
    pltpu.sync_copy(acc_sh.at[pl.ds(r0, RPT)], out_hbm.at[c, pl.ds(r0, RPT)])


def _dot(a, b):
    return jnp.dot(a, b, preferred_element_type=jnp.float32,
                   precision=lax.Precision.HIGHEST)


def _dinv_of(deg_ref):
    return lax.rsqrt(deg_ref[0, :, 0:1] + deg_ref[1, :, 0:1] + 1.0)


def _mm_body(x_ref, w_ref, o_ref):
    o_ref[...] = _dot(x_ref[...], w_ref[...])


def _prep_body(t_ref, deg_ref, o_ref, o2_ref):
    g = _dinv_of(deg_ref) * t_ref[...]
    o_ref[...] = g
    o2_ref[...] = g


def _mid_body(acc_ref, g_ref, deg_ref, b_ref, w_ref, o_ref, o2_ref):
    dinv = _dinv_of(deg_ref)
    h = jnp.maximum(dinv * (acc_ref[0] + acc_ref[1] + g_ref[...]) + b_ref[...], 0.0)
    g2 = dinv * _dot(h, w_ref[...])
    o_ref[...] = g2
    o2_ref[...] = g2


def _fin_body(acc_ref, g_ref, deg_ref, b_ref, o_ref):
    dinv = _dinv_of(deg_ref)
    o_ref[...] = dinv * (acc_ref[0] + acc_ref[1] + g_ref[...]) + b_ref[...]


_GRID = (N_PAD // BM,)
_ROWS = pl.BlockSpec((BM, D), lambda i: (i, 0))
_ACC = pl.BlockSpec((NC, BM, D), lambda i: (0, i, 0))
_DEG = pl.BlockSpec((NC, BM, D), lambda i: (0, i, 0))
_WMAT = pl.BlockSpec((D, D), lambda i: (0, 0))
_BVEC = pl.BlockSpec((1, D), lambda i: (0, 0))
_OUTF = jax.ShapeDtypeStruct((N_PAD, D), jnp.float32)

_mm = pl.pallas_call(_mm_body, grid=_GRID, in_specs=[_ROWS, _WMAT],
                     out_specs=_ROWS, out_shape=_OUTF)
_prep = pl.pallas_call(_prep_body, grid=_GRID, in_specs=[_ROWS, _DEG],
                       out_specs=[_ROWS, _ROWS], out_shape=[_OUTF, _OUTF])
_mid = pl.pallas_call(_mid_body, grid=_GRID,
                      in_specs=[_ACC, _ROWS, _DEG, _BVEC, _WMAT],
                      out_specs=[_ROWS, _ROWS], out_shape=[_OUTF, _OUTF])
_fin = pl.pallas_call(_fin_body, grid=_GRID,
                      in_specs=[_ACC, _ROWS, _DEG, _BVEC],
                      out_specs=_ROWS, out_shape=_OUTF)


def kernel(x, edge_index, W1, b1, W2, b2):
    src = edge_index[0]
    dst = edge_index[1]
    pad = jnp.full((E_PAD - E,), N, dtype=jnp.int32)
    srcp = jnp.concatenate([src, pad]).reshape(NS, NSEGT, GSEG, GCH)
    dstp = jnp.concatenate([dst, pad]).reshape(NS, NSEGT, GSEG, GCH)
    dstp_deg = dstp.reshape(NW, NCHUNK, CH)
    x_pad = jnp.pad(x, ((0, N_PAD - N), (0, 0)))
    zeros128 = jnp.zeros((N_PAD, D), jnp.float32)
    ones = jnp.ones((CH, D), jnp.float32)
    b1r = b1.reshape(1, D)
    b2r = b2.reshape(1, D)

    deg = _sc_degree(dstp_deg, zeros128, ones)
    t1 = _mm(x_pad, W1)
    g1a, g1b = _prep(t1, deg)
    acc1 = _sc_edges(g1a, g1b, srcp, dstp, zeros128)
    g2a, g2b = _mid(acc1, g1a, deg, b1r, W2)
    acc2 = _sc_edges(g2a, g2b, srcp, dstp, zeros128)
    out = _fin(acc2, g2a, deg, b2r)
    return out[:N]
